# baseline (speedup 1.0000x reference)
import dataclasses
import functools

import jax
import jax.numpy as jnp
from jax import lax
from jax.experimental import pallas as pl
from jax.experimental.pallas import tpu as pltpu
from jax.experimental.pallas import tpu_sc as plsc

N_NODES = 10000
N_EDGES = 160000
MUL = 128

_S2 = 0.7071067811865476
_S6 = 0.4082482904638631
_INV_S3 = 0.5773502691896258

_NC, _NS = 2, 16
_NW = _NC * _NS

_ROWS = 40
_RING = 5
_E1 = 76800
_E2 = N_EDGES - _E1

_HALF = N_NODES // _NC
_Z_STEP, _Z_ROWS = 312, 320
_Z_CHUNK = 80


def _mesh():
    return plsc.VectorSubcoreMesh(core_axis_name="c", subcore_axis_name="s",
                                  num_cores=_NC, num_subcores=_NS)


@functools.cache
def _make_sc_gather(n_edges):
    n_b = n_edges // (_NW * _ROWS)

    @functools.partial(
        pl.kernel,
        out_type=jax.ShapeDtypeStruct((n_edges, 4 * MUL), jnp.float32),
        mesh=_mesh(),
        scratch_types=[
            pltpu.VMEM((n_b, _ROWS), jnp.int32),
            [pltpu.VMEM((_ROWS, 4 * MUL), jnp.float32)
             for _ in range(_RING)],
            [pltpu.SemaphoreType.DMA for _ in range(_RING)],
            [pltpu.SemaphoreType.DMA for _ in range(_RING)],
        ],
    )
    def _sc_gather(nf_hbm, src_hbm, h_hbm, idxb, rbufs, gsems, wsems):
        c = lax.axis_index("c")
        s = lax.axis_index("s")
        w = c * _NS + s
        pltpu.sync_copy(src_hbm.at[w], idxb)
        base = w * n_b * _ROWS

        def gath(j, b):
            return pltpu.async_copy(nf_hbm.at[idxb.at[b]], rbufs[j],
                                    gsems[j])

        for j in range(_RING):
            gath(j, j)

        @pl.loop(0, n_b // _RING)
        def _(g):
            for j in range(_RING):
                pltpu.make_async_copy(nf_hbm.at[idxb.at[0]], rbufs[j],
                                      gsems[j]).wait()
                pltpu.async_copy(
                    rbufs[j],
                    h_hbm.at[pl.ds(base + (g * _RING + j) * _ROWS, _ROWS)],
                    wsems[j])
            for j in range(_RING):
                pltpu.make_async_copy(
                    rbufs[j], h_hbm.at[pl.ds(0, _ROWS)], wsems[j]).wait()

                @pl.when(g < n_b // _RING - 1)
                def _():
                    gath(j, (g + 1) * _RING + j)

    return _sc_gather


_E_BLK = 640


def _tc_body(h_ref, ang_ref, w_ref, out_ref):
    h = h_ref[...]
    ang = ang_ref[...]
    w = w_ref[...]
    h0 = h[:, 0 * MUL:1 * MUL]
    h1y = h[:, 1 * MUL:2 * MUL]
    h1z = h[:, 2 * MUL:3 * MUL]
    h1x = h[:, 3 * MUL:4 * MUL]
    y0 = ang[:, 0:1]
    y1y = ang[:, 1:2]
    y1z = ang[:, 2:3]
    y1x = ang[:, 3:4]
    w0 = w[:, 0 * MUL:1 * MUL]
    w1 = w[:, 1 * MUL:2 * MUL]
    w2 = w[:, 2 * MUL:3 * MUL]
    w3 = w[:, 3 * MUL:4 * MUL]
    w4 = w[:, 4 * MUL:5 * MUL]

    dot = h1y * y1y + h1z * y1z + h1x * y1x
    out_ref[0] = w0 * h0 * y0 + w3 * dot * _INV_S3
    out_ref[1] = w1 * h0 * y1y + w2 * h1y * y0
    out_ref[2] = w1 * h0 * y1z + w2 * h1z * y0
    out_ref[3] = w1 * h0 * y1x + w2 * h1x * y0
    out_ref[4] = w4 * (_S2 * (h1x * y1y + h1y * y1x))
    out_ref[5] = w4 * (_S2 * (h1y * y1z + h1z * y1y))
    out_ref[6] = w4 * (2.0 * _S6 * h1z * y1z - _S6 * (h1x * y1x + h1y * y1y))
    out_ref[7] = w4 * (_S2 * (h1x * y1z + h1z * y1x))
    out_ref[8] = w4 * (_S2 * (h1x * y1x - h1y * y1y))


def _tc_compute(h, ang, w, off):
    n_edges = h.shape[0]
    ob = off // _E_BLK
    return pl.pallas_call(
        _tc_body,
        grid=(n_edges // _E_BLK,),
        in_specs=[
            pl.BlockSpec((_E_BLK, 4 * MUL), lambda i: (i, 0)),
            pl.BlockSpec((_E_BLK, 4), lambda i: (i + ob, 0)),
            pl.BlockSpec((_E_BLK, 5 * MUL), lambda i: (i + ob, 0)),
        ],
        out_specs=pl.BlockSpec((9, _E_BLK, MUL), lambda i: (0, i, 0)),
        out_shape=jax.ShapeDtypeStruct((9, n_edges, MUL), jnp.float32),
    )(h, ang, w)


_C_ROWS = 16
_S_RING = 10


@functools.cache
def _make_sc_scatter(n_edges, init):
    e_t = n_edges // _NS
    n_g = e_t // 16
    n_pad = e_t + 160

    out_t = jax.ShapeDtypeStruct((N_NODES, 9 * MUL), jnp.float32)
    scratch = [
        pltpu.VMEM_SHARED((_HALF + 8, MUL), jnp.float32),
        pltpu.VMEM((_Z_CHUNK, MUL), jnp.float32),
        [pltpu.VMEM((_C_ROWS, MUL), jnp.float32) for _ in range(_S_RING)],
        pltpu.VMEM((e_t,), jnp.int32),
        pltpu.VMEM((n_pad,), jnp.int32),
        pltpu.VMEM((n_pad,), jnp.int32),
        [pltpu.SemaphoreType.DMA for _ in range(_S_RING)],
        [pltpu.SemaphoreType.DMA for _ in range(_S_RING)],
    ]

    def _body(msgs_hbm, dst_hbm, prev_hbm, out_hbm, acc, zbuf, mbufs, rawd,
              dstc, elist, rsems, asems):
        c = lax.axis_index("c")
        s = lax.axis_index("s")
        zero16 = jnp.zeros((16,), jnp.float32)

        @pl.loop(0, _Z_CHUNK)
        def _(i):
            @pl.loop(0, MUL // 16)
            def _(j):
                zbuf[i, pl.ds(j * 16, 16)] = zero16

        pltpu.sync_copy(dst_hbm.at[s], rawd)

        base = c * _HALF
        ebase = s * e_t
        iota16 = lax.iota(jnp.int32, 16)

        def comp_body(g, cnt):
            v = rawd[pl.ds(g * 16, 16)] - base
            ok = jnp.logical_and(v >= 0, v < _HALF)
            plsc.store_compressed(dstc.at[pl.ds(cnt, 16)], v, mask=ok)
            plsc.store_compressed(elist.at[pl.ds(cnt, 16)],
                                  iota16 + (ebase + g * 16), mask=ok)
            npop = jnp.max(plsc.all_reduce_population_count(ok))
            return cnt + npop

        cnt = lax.fori_loop(0, n_g, comp_body, jnp.int32(0))

        junk_d = jnp.full((16,), _HALF, jnp.int32)
        junk_e = jnp.full((16,), ebase, jnp.int32)
        for k in range(10):
            dstc[pl.ds(cnt + k * 16, 16)] = junk_d
            elist[pl.ds(cnt + k * 16, 16)] = junk_e

        n_it = (cnt + 159) // 160
        grow = base + s * _Z_STEP

        for chunk in range(9):
            if init:
                pltpu.sync_copy(
                    prev_hbm.at[pl.ds(grow, _Z_ROWS),
                                pl.ds(chunk * MUL, MUL)],
                    acc.at[pl.ds(s * _Z_STEP, _Z_ROWS)])
            else:
                for k in range(_Z_ROWS // _Z_CHUNK):
                    pltpu.sync_copy(
                        zbuf,
                        acc.at[pl.ds(s * _Z_STEP + k * _Z_CHUNK, _Z_CHUNK)])
            plsc.subcore_barrier()

            mrows = msgs_hbm.at[chunk]

            def read(j, b):
                return pltpu.async_copy(
                    mrows.at[elist.at[pl.ds(b * _C_ROWS, _C_ROWS)]],
                    mbufs[j], rsems[j])

            def wait_read(j):
                pltpu.make_async_copy(
                    mrows.at[pl.ds(0, _C_ROWS)], mbufs[j], rsems[j]).wait()

            def add(j, b):
                dvec = dstc[pl.ds(b * _C_ROWS, _C_ROWS)]
                pltpu.async_copy(mbufs[j], acc.at[dvec],
                                 asems[j], add=True)

            def wait_add(j):
                pltpu.make_async_copy(
                    mrows.at[pl.ds(0, _C_ROWS)], mbufs[j], asems[j]).wait()

            for j in range(_S_RING):
                read(j, j)

            @pl.loop(0, n_it - 1)
            def _(g):
                b0 = g * _S_RING
                for j in range(_S_RING):
                    wait_read(j)
                    add(j, b0 + j)
                for j in range(_S_RING):
                    wait_add(j)
                    read(j, b0 + _S_RING + j)

            b0 = (n_it - 1) * _S_RING
            for j in range(_S_RING):
                wait_read(j)
                add(j, b0 + j)
            for j in range(_S_RING):
                wait_add(j)

            plsc.subcore_barrier()
            pltpu.sync_copy(
                acc.at[pl.ds(s * _Z_STEP, _Z_ROWS)],
                out_hbm.at[pl.ds(grow, _Z_ROWS), pl.ds(chunk * MUL, MUL)])
            plsc.subcore_barrier()

    cp = pltpu.CompilerParams()
    if "needs_layout_passes" in pltpu.CompilerParams.__dataclass_fields__:
        cp = dataclasses.replace(cp, needs_layout_passes=False)

    if init:
        return functools.partial(pl.kernel, out_type=out_t, mesh=_mesh(),
                                 scratch_types=scratch,
                                 compiler_params=cp)(_body)

    def _body_noprev(msgs_hbm, dst_hbm, out_hbm, *rest):
        return _body(msgs_hbm, dst_hbm, None, out_hbm, *rest)

    return functools.partial(pl.kernel, out_type=out_t, mesh=_mesh(),
                             scratch_types=scratch,
                             compiler_params=cp)(_body_noprev)


def kernel(node_features, edge_angular, edge_index, tp_weights):
    nb1 = _E1 // (_NW * _ROWS)
    nb2 = _E2 // (_NW * _ROWS)

    src1 = edge_index[:_E1, 0].reshape(_NW, nb1, _ROWS)
    src2 = edge_index[_E1:, 0].reshape(_NW, nb2, _ROWS)
    dst1 = edge_index[:_E1, 1].reshape(_NS, _E1 // _NS)
    dst2 = edge_index[_E1:, 1].reshape(_NS, _E2 // _NS)

    h1 = _make_sc_gather(_E1)(node_features, src1)
    m1 = _tc_compute(h1, edge_angular, tp_weights, 0)
    h2 = _make_sc_gather(_E2)(node_features, src2)
    m2 = _tc_compute(h2, edge_angular, tp_weights, _E1)
    o1 = _make_sc_scatter(_E1, False)(m1, dst1)
    return _make_sc_scatter(_E2, True)(m2, dst2, o1)


# speedup vs baseline: 1.0196x; 1.0196x over previous
import dataclasses
import functools

import jax
import jax.numpy as jnp
from jax import lax
from jax.experimental import pallas as pl
from jax.experimental.pallas import tpu as pltpu
from jax.experimental.pallas import tpu_sc as plsc

N_NODES = 10000
N_EDGES = 160000
MUL = 128

_S2 = 0.7071067811865476
_S6 = 0.4082482904638631
_INV_S3 = 0.5773502691896258

_NC, _NS = 2, 16
_NW = _NC * _NS

_ROWS = 40
_RING = 5
_E1 = 76800
_E2 = N_EDGES - _E1

_HALF = N_NODES // _NC
_Z_STEP, _Z_ROWS = 312, 320
_Z_CHUNK = 160


def _mesh():
    return plsc.VectorSubcoreMesh(core_axis_name="c", subcore_axis_name="s",
                                  num_cores=_NC, num_subcores=_NS)


@functools.cache
def _make_sc_gather(n_edges):
    n_b = n_edges // (_NW * _ROWS)

    @functools.partial(
        pl.kernel,
        out_type=jax.ShapeDtypeStruct((n_edges, 4 * MUL), jnp.float32),
        mesh=_mesh(),
        scratch_types=[
            pltpu.VMEM((n_b, _ROWS), jnp.int32),
            [pltpu.VMEM((_ROWS, 4 * MUL), jnp.float32)
             for _ in range(_RING)],
            [pltpu.SemaphoreType.DMA for _ in range(_RING)],
            [pltpu.SemaphoreType.DMA for _ in range(_RING)],
        ],
    )
    def _sc_gather(nf_hbm, src_hbm, h_hbm, idxb, rbufs, gsems, wsems):
        c = lax.axis_index("c")
        s = lax.axis_index("s")
        w = c * _NS + s
        pltpu.sync_copy(src_hbm.at[w], idxb)
        base = w * n_b * _ROWS

        def gath(j, b):
            return pltpu.async_copy(nf_hbm.at[idxb.at[b]], rbufs[j],
                                    gsems[j])

        for j in range(_RING):
            gath(j, j)

        @pl.loop(0, n_b // _RING)
        def _(g):
            for j in range(_RING):
                pltpu.make_async_copy(nf_hbm.at[idxb.at[0]], rbufs[j],
                                      gsems[j]).wait()
                pltpu.async_copy(
                    rbufs[j],
                    h_hbm.at[pl.ds(base + (g * _RING + j) * _ROWS, _ROWS)],
                    wsems[j])
            for j in range(_RING):
                pltpu.make_async_copy(
                    rbufs[j], h_hbm.at[pl.ds(0, _ROWS)], wsems[j]).wait()

                @pl.when(g < n_b // _RING - 1)
                def _():
                    gath(j, (g + 1) * _RING + j)

    return _sc_gather


_E_BLK = 640


def _tc_body(h_ref, ang_ref, w_ref, out_ref):
    h = h_ref[...]
    ang = ang_ref[...]
    w = w_ref[...]
    h0 = h[:, 0 * MUL:1 * MUL]
    h1y = h[:, 1 * MUL:2 * MUL]
    h1z = h[:, 2 * MUL:3 * MUL]
    h1x = h[:, 3 * MUL:4 * MUL]
    y0 = ang[:, 0:1]
    y1y = ang[:, 1:2]
    y1z = ang[:, 2:3]
    y1x = ang[:, 3:4]
    w0 = w[:, 0 * MUL:1 * MUL]
    w1 = w[:, 1 * MUL:2 * MUL]
    w2 = w[:, 2 * MUL:3 * MUL]
    w3 = w[:, 3 * MUL:4 * MUL]
    w4 = w[:, 4 * MUL:5 * MUL]

    dot = h1y * y1y + h1z * y1z + h1x * y1x
    out_ref[0] = w0 * h0 * y0 + w3 * dot * _INV_S3
    out_ref[1] = w1 * h0 * y1y + w2 * h1y * y0
    out_ref[2] = w1 * h0 * y1z + w2 * h1z * y0
    out_ref[3] = w1 * h0 * y1x + w2 * h1x * y0
    out_ref[4] = w4 * (_S2 * (h1x * y1y + h1y * y1x))
    out_ref[5] = w4 * (_S2 * (h1y * y1z + h1z * y1y))
    out_ref[6] = w4 * (2.0 * _S6 * h1z * y1z - _S6 * (h1x * y1x + h1y * y1y))
    out_ref[7] = w4 * (_S2 * (h1x * y1z + h1z * y1x))
    out_ref[8] = w4 * (_S2 * (h1x * y1x - h1y * y1y))


def _tc_compute(h, ang, w, off):
    n_edges = h.shape[0]
    ob = off // _E_BLK
    return pl.pallas_call(
        _tc_body,
        grid=(n_edges // _E_BLK,),
        in_specs=[
            pl.BlockSpec((_E_BLK, 4 * MUL), lambda i: (i, 0)),
            pl.BlockSpec((_E_BLK, 4), lambda i: (i + ob, 0)),
            pl.BlockSpec((_E_BLK, 5 * MUL), lambda i: (i + ob, 0)),
        ],
        out_specs=pl.BlockSpec((9, _E_BLK, MUL), lambda i: (0, i, 0)),
        out_shape=jax.ShapeDtypeStruct((9, n_edges, MUL), jnp.float32),
    )(h, ang, w)


_C_ROWS = 16
_S_RING = 5


@functools.cache
def _make_sc_scatter(n_edges, init):
    e_t = n_edges // _NS
    n_g = e_t // 16
    n_pad = e_t + 160

    out_t = jax.ShapeDtypeStruct((N_NODES, 9 * MUL), jnp.float32)
    scratch = [
        pltpu.VMEM_SHARED((_HALF + 8, MUL), jnp.float32),
        pltpu.VMEM((_Z_CHUNK, MUL), jnp.float32),
        [pltpu.VMEM((_C_ROWS, MUL), jnp.float32) for _ in range(_S_RING)],
        pltpu.VMEM((e_t,), jnp.int32),
        pltpu.VMEM((n_pad,), jnp.int32),
        pltpu.VMEM((n_pad,), jnp.int32),
        [pltpu.SemaphoreType.DMA for _ in range(_S_RING)],
        [pltpu.SemaphoreType.DMA for _ in range(_S_RING)],
    ]

    def _body(msgs_hbm, dst_hbm, prev_hbm, out_hbm, acc, zbuf, mbufs, rawd,
              dstc, elist, rsems, asems):
        c = lax.axis_index("c")
        s = lax.axis_index("s")
        zero16 = jnp.zeros((16,), jnp.float32)

        @pl.loop(0, _Z_CHUNK)
        def _(i):
            @pl.loop(0, MUL // 16)
            def _(j):
                zbuf[i, pl.ds(j * 16, 16)] = zero16

        pltpu.sync_copy(dst_hbm.at[s], rawd)

        base = c * _HALF
        ebase = s * e_t
        iota16 = lax.iota(jnp.int32, 16)

        def comp_body(g, cnt):
            v = rawd[pl.ds(g * 16, 16)] - base
            ok = jnp.logical_and(v >= 0, v < _HALF)
            plsc.store_compressed(dstc.at[pl.ds(cnt, 16)], v, mask=ok)
            plsc.store_compressed(elist.at[pl.ds(cnt, 16)],
                                  iota16 + (ebase + g * 16), mask=ok)
            npop = jnp.max(plsc.all_reduce_population_count(ok))
            return cnt + npop

        cnt = lax.fori_loop(0, n_g, comp_body, jnp.int32(0))

        junk_d = jnp.full((16,), _HALF, jnp.int32)
        junk_e = jnp.full((16,), ebase, jnp.int32)
        for k in range(10):
            dstc[pl.ds(cnt + k * 16, 16)] = junk_d
            elist[pl.ds(cnt + k * 16, 16)] = junk_e

        n_it = jnp.maximum(1, (cnt + 79) // 80)
        grow = base + s * _Z_STEP

        for chunk in range(9):
            if init:
                pltpu.sync_copy(
                    prev_hbm.at[pl.ds(grow, _Z_ROWS),
                                pl.ds(chunk * MUL, MUL)],
                    acc.at[pl.ds(s * _Z_STEP, _Z_ROWS)])
            else:
                for k in range(_Z_ROWS // _Z_CHUNK):
                    pltpu.sync_copy(
                        zbuf,
                        acc.at[pl.ds(s * _Z_STEP + k * _Z_CHUNK, _Z_CHUNK)])
            plsc.subcore_barrier()

            mrows = msgs_hbm.at[chunk]

            def read(j, b):
                return pltpu.async_copy(
                    mrows.at[elist.at[pl.ds(b * _C_ROWS, _C_ROWS)]],
                    mbufs[j], rsems[j])

            def wait_read(j):
                pltpu.make_async_copy(
                    mrows.at[pl.ds(0, _C_ROWS)], mbufs[j], rsems[j]).wait()

            def add(j, b):
                dvec = dstc[pl.ds(b * _C_ROWS, _C_ROWS)]
                pltpu.async_copy(mbufs[j], acc.at[dvec],
                                 asems[j], add=True)

            def wait_add(j):
                pltpu.make_async_copy(
                    mrows.at[pl.ds(0, _C_ROWS)], mbufs[j], asems[j]).wait()

            for j in range(_S_RING):
                read(j, j)

            @pl.loop(0, n_it - 1)
            def _(g):
                b0 = g * _S_RING
                for j in range(_S_RING):
                    wait_read(j)
                    add(j, b0 + j)
                for j in range(_S_RING):
                    wait_add(j)
                    read(j, b0 + _S_RING + j)

            b0 = (n_it - 1) * _S_RING
            for j in range(_S_RING):
                wait_read(j)
                add(j, b0 + j)
            for j in range(_S_RING):
                wait_add(j)

            plsc.subcore_barrier()
            pltpu.sync_copy(
                acc.at[pl.ds(s * _Z_STEP, _Z_ROWS)],
                out_hbm.at[pl.ds(grow, _Z_ROWS), pl.ds(chunk * MUL, MUL)])
            plsc.subcore_barrier()

    cp = pltpu.CompilerParams()
    if "needs_layout_passes" in pltpu.CompilerParams.__dataclass_fields__:
        cp = dataclasses.replace(cp, needs_layout_passes=False)

    if init:
        return functools.partial(pl.kernel, out_type=out_t, mesh=_mesh(),
                                 scratch_types=scratch,
                                 compiler_params=cp)(_body)

    def _body_noprev(msgs_hbm, dst_hbm, out_hbm, *rest):
        return _body(msgs_hbm, dst_hbm, None, out_hbm, *rest)

    return functools.partial(pl.kernel, out_type=out_t, mesh=_mesh(),
                             scratch_types=scratch,
                             compiler_params=cp)(_body_noprev)


def kernel(node_features, edge_angular, edge_index, tp_weights):
    nb1 = _E1 // (_NW * _ROWS)
    nb2 = _E2 // (_NW * _ROWS)

    src1 = edge_index[:_E1, 0].reshape(_NW, nb1, _ROWS)
    src2 = edge_index[_E1:, 0].reshape(_NW, nb2, _ROWS)
    dst1 = edge_index[:_E1, 1].reshape(_NS, _E1 // _NS)
    dst2 = edge_index[_E1:, 1].reshape(_NS, _E2 // _NS)

    h1 = _make_sc_gather(_E1)(node_features, src1)
    m1 = _tc_compute(h1, edge_angular, tp_weights, 0)
    h2 = _make_sc_gather(_E2)(node_features, src2)
    m2 = _tc_compute(h2, edge_angular, tp_weights, _E1)
    o1 = _make_sc_scatter(_E1, False)(m1, dst1)
    return _make_sc_scatter(_E2, True)(m2, dst2, o1)


# speedup vs baseline: 1.0514x; 1.0312x over previous
import dataclasses
import functools

import jax
import jax.numpy as jnp
from jax import lax
from jax.experimental import pallas as pl
from jax.experimental.pallas import tpu as pltpu
from jax.experimental.pallas import tpu_sc as plsc

N_NODES = 10000
N_EDGES = 160000
MUL = 128

_S2 = 0.7071067811865476
_S6 = 0.4082482904638631
_INV_S3 = 0.5773502691896258

_NC, _NS = 2, 16
_NW = _NC * _NS

_ROWS = 40
_RING = 5
_E1 = 76800
_E2 = N_EDGES - _E1

_HALF = N_NODES // _NC
_Z_STEP, _Z_ROWS = 312, 320
_Z_CHUNK = 80


def _mesh():
    return plsc.VectorSubcoreMesh(core_axis_name="c", subcore_axis_name="s",
                                  num_cores=_NC, num_subcores=_NS)


@functools.cache
def _make_sc_gather(n_edges):
    n_b = n_edges // (_NW * _ROWS)

    @functools.partial(
        pl.kernel,
        out_type=jax.ShapeDtypeStruct((n_edges, 4 * MUL), jnp.float32),
        mesh=_mesh(),
        scratch_types=[
            pltpu.VMEM((n_b, _ROWS), jnp.int32),
            [pltpu.VMEM((_ROWS, 4 * MUL), jnp.float32)
             for _ in range(_RING)],
            [pltpu.SemaphoreType.DMA for _ in range(_RING)],
            [pltpu.SemaphoreType.DMA for _ in range(_RING)],
        ],
    )
    def _sc_gather(nf_hbm, src_hbm, h_hbm, idxb, rbufs, gsems, wsems):
        c = lax.axis_index("c")
        s = lax.axis_index("s")
        w = c * _NS + s
        pltpu.sync_copy(src_hbm.at[w], idxb)
        base = w * n_b * _ROWS

        def gath(j, b):
            return pltpu.async_copy(nf_hbm.at[idxb.at[b]], rbufs[j],
                                    gsems[j])

        for j in range(_RING):
            gath(j, j)

        @pl.loop(0, n_b // _RING)
        def _(g):
            for j in range(_RING):
                pltpu.make_async_copy(nf_hbm.at[idxb.at[0]], rbufs[j],
                                      gsems[j]).wait()
                pltpu.async_copy(
                    rbufs[j],
                    h_hbm.at[pl.ds(base + (g * _RING + j) * _ROWS, _ROWS)],
                    wsems[j])
            for j in range(_RING):
                pltpu.make_async_copy(
                    rbufs[j], h_hbm.at[pl.ds(0, _ROWS)], wsems[j]).wait()

                @pl.when(g < n_b // _RING - 1)
                def _():
                    gath(j, (g + 1) * _RING + j)

    return _sc_gather


_E_BLK = 640


def _tc_body(h_ref, ang_ref, w_ref, out_ref):
    h = h_ref[...]
    ang = ang_ref[...]
    w = w_ref[...]
    h0 = h[:, 0 * MUL:1 * MUL]
    h1y = h[:, 1 * MUL:2 * MUL]
    h1z = h[:, 2 * MUL:3 * MUL]
    h1x = h[:, 3 * MUL:4 * MUL]
    y0 = ang[:, 0:1]
    y1y = ang[:, 1:2]
    y1z = ang[:, 2:3]
    y1x = ang[:, 3:4]
    w0 = w[:, 0 * MUL:1 * MUL]
    w1 = w[:, 1 * MUL:2 * MUL]
    w2 = w[:, 2 * MUL:3 * MUL]
    w3 = w[:, 3 * MUL:4 * MUL]
    w4 = w[:, 4 * MUL:5 * MUL]

    dot = h1y * y1y + h1z * y1z + h1x * y1x
    out_ref[0] = w0 * h0 * y0 + w3 * dot * _INV_S3
    out_ref[1] = w1 * h0 * y1y + w2 * h1y * y0
    out_ref[2] = w1 * h0 * y1z + w2 * h1z * y0
    out_ref[3] = w1 * h0 * y1x + w2 * h1x * y0
    out_ref[4] = w4 * (_S2 * (h1x * y1y + h1y * y1x))
    out_ref[5] = w4 * (_S2 * (h1y * y1z + h1z * y1y))
    out_ref[6] = w4 * (2.0 * _S6 * h1z * y1z - _S6 * (h1x * y1x + h1y * y1y))
    out_ref[7] = w4 * (_S2 * (h1x * y1z + h1z * y1x))
    out_ref[8] = w4 * (_S2 * (h1x * y1x - h1y * y1y))


def _tc_compute(h, ang, w, off):
    n_edges = h.shape[0]
    ob = off // _E_BLK
    return pl.pallas_call(
        _tc_body,
        grid=(n_edges // _E_BLK,),
        in_specs=[
            pl.BlockSpec((_E_BLK, 4 * MUL), lambda i: (i, 0)),
            pl.BlockSpec((_E_BLK, 4), lambda i: (i + ob, 0)),
            pl.BlockSpec((_E_BLK, 5 * MUL), lambda i: (i + ob, 0)),
        ],
        out_specs=pl.BlockSpec((9, _E_BLK, MUL), lambda i: (0, i, 0)),
        out_shape=jax.ShapeDtypeStruct((9, n_edges, MUL), jnp.float32),
    )(h, ang, w)


_C_ROWS = 32
_S_RING = 5


@functools.cache
def _make_sc_scatter(n_edges, init):
    e_t = n_edges // _NS
    n_g = e_t // 16
    n_pad = e_t + 160

    out_t = jax.ShapeDtypeStruct((N_NODES, 9 * MUL), jnp.float32)
    scratch = [
        pltpu.VMEM_SHARED((_HALF + 8, MUL), jnp.float32),
        pltpu.VMEM((_Z_CHUNK, MUL), jnp.float32),
        [pltpu.VMEM((_C_ROWS, MUL), jnp.float32) for _ in range(_S_RING)],
        pltpu.VMEM((e_t,), jnp.int32),
        pltpu.VMEM((n_pad,), jnp.int32),
        pltpu.VMEM((n_pad,), jnp.int32),
        [pltpu.SemaphoreType.DMA for _ in range(_S_RING)],
        [pltpu.SemaphoreType.DMA for _ in range(_S_RING)],
        [pltpu.SemaphoreType.DMA for _ in range(_S_RING)],
    ]

    def _body(msgs_hbm, dst_hbm, prev_hbm, out_hbm, acc, zbuf, mbufs, rawd,
              dstc, elist, rsems, asemsa, asemsb):
        c = lax.axis_index("c")
        s = lax.axis_index("s")
        zero16 = jnp.zeros((16,), jnp.float32)

        @pl.loop(0, _Z_CHUNK)
        def _(i):
            @pl.loop(0, MUL // 16)
            def _(j):
                zbuf[i, pl.ds(j * 16, 16)] = zero16

        pltpu.sync_copy(dst_hbm.at[s], rawd)

        base = c * _HALF
        ebase = s * e_t
        iota16 = lax.iota(jnp.int32, 16)

        def comp_body(g, cnt):
            v = rawd[pl.ds(g * 16, 16)] - base
            ok = jnp.logical_and(v >= 0, v < _HALF)
            plsc.store_compressed(dstc.at[pl.ds(cnt, 16)], v, mask=ok)
            plsc.store_compressed(elist.at[pl.ds(cnt, 16)],
                                  iota16 + (ebase + g * 16), mask=ok)
            npop = jnp.max(plsc.all_reduce_population_count(ok))
            return cnt + npop

        cnt = lax.fori_loop(0, n_g, comp_body, jnp.int32(0))

        junk_d = jnp.full((16,), _HALF, jnp.int32)
        junk_e = jnp.full((16,), ebase, jnp.int32)
        for k in range(10):
            dstc[pl.ds(cnt + k * 16, 16)] = junk_d
            elist[pl.ds(cnt + k * 16, 16)] = junk_e

        n_it = jnp.maximum(1, (cnt + 159) // 160)
        grow = base + s * _Z_STEP

        for chunk in range(9):
            if init:
                pltpu.sync_copy(
                    prev_hbm.at[pl.ds(grow, _Z_ROWS),
                                pl.ds(chunk * MUL, MUL)],
                    acc.at[pl.ds(s * _Z_STEP, _Z_ROWS)])
            else:
                for k in range(_Z_ROWS // _Z_CHUNK):
                    pltpu.sync_copy(
                        zbuf,
                        acc.at[pl.ds(s * _Z_STEP + k * _Z_CHUNK, _Z_CHUNK)])
            plsc.subcore_barrier()

            mrows = msgs_hbm.at[chunk]

            def read(j, b):
                return pltpu.async_copy(
                    mrows.at[elist.at[pl.ds(b * _C_ROWS, _C_ROWS)]],
                    mbufs[j], rsems[j])

            def wait_read(j):
                pltpu.make_async_copy(
                    mrows.at[pl.ds(0, _C_ROWS)], mbufs[j], rsems[j]).wait()

            def add(j, b):
                dveca = dstc[pl.ds(b * _C_ROWS, 16)]
                dvecb = dstc[pl.ds(b * _C_ROWS + 16, 16)]
                pltpu.async_copy(mbufs[j].at[pl.ds(0, 16)], acc.at[dveca],
                                 asemsa[j], add=True)
                pltpu.async_copy(mbufs[j].at[pl.ds(16, 16)], acc.at[dvecb],
                                 asemsb[j], add=True)

            def wait_add(j):
                pltpu.make_async_copy(
                    mrows.at[pl.ds(0, 16)], mbufs[j].at[pl.ds(0, 16)],
                    asemsa[j]).wait()
                pltpu.make_async_copy(
                    mrows.at[pl.ds(0, 16)], mbufs[j].at[pl.ds(16, 16)],
                    asemsb[j]).wait()

            for j in range(_S_RING):
                read(j, j)

            @pl.loop(0, n_it - 1)
            def _(g):
                b0 = g * _S_RING
                for j in range(_S_RING):
                    wait_read(j)
                    add(j, b0 + j)
                for j in range(_S_RING):
                    wait_add(j)
                    read(j, b0 + _S_RING + j)

            b0 = (n_it - 1) * _S_RING
            for j in range(_S_RING):
                wait_read(j)
                add(j, b0 + j)
            for j in range(_S_RING):
                wait_add(j)

            plsc.subcore_barrier()
            pltpu.sync_copy(
                acc.at[pl.ds(s * _Z_STEP, _Z_ROWS)],
                out_hbm.at[pl.ds(grow, _Z_ROWS), pl.ds(chunk * MUL, MUL)])
            plsc.subcore_barrier()

    cp = pltpu.CompilerParams()
    if "needs_layout_passes" in pltpu.CompilerParams.__dataclass_fields__:
        cp = dataclasses.replace(cp, needs_layout_passes=False)

    if init:
        return functools.partial(pl.kernel, out_type=out_t, mesh=_mesh(),
                                 scratch_types=scratch,
                                 compiler_params=cp)(_body)

    def _body_noprev(msgs_hbm, dst_hbm, out_hbm, *rest):
        return _body(msgs_hbm, dst_hbm, None, out_hbm, *rest)

    return functools.partial(pl.kernel, out_type=out_t, mesh=_mesh(),
                             scratch_types=scratch,
                             compiler_params=cp)(_body_noprev)


def kernel(node_features, edge_angular, edge_index, tp_weights):
    nb1 = _E1 // (_NW * _ROWS)
    nb2 = _E2 // (_NW * _ROWS)

    src1 = edge_index[:_E1, 0].reshape(_NW, nb1, _ROWS)
    src2 = edge_index[_E1:, 0].reshape(_NW, nb2, _ROWS)
    dst1 = edge_index[:_E1, 1].reshape(_NS, _E1 // _NS)
    dst2 = edge_index[_E1:, 1].reshape(_NS, _E2 // _NS)

    h1 = _make_sc_gather(_E1)(node_features, src1)
    m1 = _tc_compute(h1, edge_angular, tp_weights, 0)
    h2 = _make_sc_gather(_E2)(node_features, src2)
    m2 = _tc_compute(h2, edge_angular, tp_weights, _E1)
    o1 = _make_sc_scatter(_E1, False)(m1, dst1)
    return _make_sc_scatter(_E2, True)(m2, dst2, o1)
